# Initial kernel scaffold; baseline (speedup 1.0000x reference)
import jax
import jax.numpy as jnp
from jax.experimental import pallas as pl


def kernel(atomic_numbers, pos, edge_index, atom_emb, W_rbf, W_edge_in, W1, Ws, W2, W3, W_E, W_F):
    raise NotImplementedError("write your pallas kernel here")


# speedup vs baseline: 1.9594x; 1.9594x over previous
import functools

import jax
import jax.numpy as jnp
from jax import lax
from jax.experimental import pallas as pl
from jax.experimental.pallas import tpu as pltpu
from jax.experimental.pallas import tpu_sc as plsc

N = 50000
E = 800000
NUM_RADIAL = 64
EMB_RBF = 16
EMB_ATOM = 64
EMB_EDGE = 32
NUM_BLOCKS = 4
CUTOFF = 6.0

NC = 2
NS = 16
NW = NC * NS
CH = 128

NP = 53248
EP = 802816
TE = 4096
TN = 4096
GE = EP // TE
GN = NP // TN

def _silu(x):
    return x / (1.0 + jnp.exp(-x))


def _sc_gather(table, idx, d):
    b = idx.shape[0]
    per_w = b // NW
    n_chunks = per_w // CH

    @functools.partial(
        pl.kernel,
        out_type=jax.ShapeDtypeStruct((b, d), jnp.float32),
        mesh=plsc.VectorSubcoreMesh(core_axis_name="c", subcore_axis_name="s"),
        compiler_params=pltpu.CompilerParams(use_tc_tiling_on_sc=False),
        scratch_types=[
            pltpu.VMEM((CH,), jnp.int32),
            pltpu.VMEM((CH, d), jnp.float32),
            pltpu.SemaphoreType.DMA,
        ],
    )
    def k(table_h, idx_h, out_h, idx_v, rows_v, sem):
        w = lax.axis_index("s") * NC + lax.axis_index("c")
        base = w * per_w

        def chunk(i, carry):
            off = base + i * CH
            pltpu.sync_copy(idx_h.at[pl.ds(off, CH)], idx_v)
            pltpu.async_copy(table_h.at[idx_v], rows_v, sem).wait()
            pltpu.sync_copy(rows_v, out_h.at[pl.ds(off, CH)])
            return carry

        lax.fori_loop(0, n_chunks, chunk, 0)

    return k(table, idx)


def _sc_scatter_add(vals, idx, zeros, d):
    half = EP // NC
    per_t = half // NS
    n_chunks = per_t // CH
    rows_t = NP // NS

    @functools.partial(
        pl.kernel,
        out_type=jax.ShapeDtypeStruct((NC, NP, d), jnp.float32),
        mesh=plsc.VectorSubcoreMesh(core_axis_name="c", subcore_axis_name="s"),
        compiler_params=pltpu.CompilerParams(use_tc_tiling_on_sc=False),
        scratch_types=[
            pltpu.VMEM((CH,), jnp.int32),
            pltpu.VMEM((CH, d), jnp.float32),
            pltpu.VMEM_SHARED((NP, d), jnp.float32),
            pltpu.SemaphoreType.DMA,
        ],
    )
    def k(vals_h, idx_h, zeros_h, out_h, idx_v, vals_v, acc, sem):
        c = lax.axis_index("c")
        s = lax.axis_index("s")
        r0 = s * rows_t
        pltpu.sync_copy(zeros_h.at[pl.ds(r0, rows_t)], acc.at[pl.ds(r0, rows_t)])
        plsc.subcore_barrier()
        base = c * half + s * per_t

        def chunk(i, carry):
            off = base + i * CH
            pltpu.sync_copy(idx_h.at[pl.ds(off, CH)], idx_v)
            pltpu.sync_copy(vals_h.at[pl.ds(off, CH)], vals_v)
            pltpu.sync_copy(vals_v, acc.at[idx_v], add=True)
            return carry

        lax.fori_loop(0, n_chunks, chunk, 0)
        plsc.subcore_barrier()
        pltpu.sync_copy(acc.at[pl.ds(r0, rows_t)], out_h.at[c, pl.ds(r0, rows_t)])

    return k(vals, idx, zeros)


def _dot(a, b):
    return jnp.dot(a, b, preferred_element_type=jnp.float32,
                   precision=lax.Precision.HIGHEST)


def _tc_atom_init(h0, wa, wb):
    def body(h_ref, wa_ref, wb_ref, a_ref, b_ref):
        h = h_ref[...]
        a_ref[...] = _dot(h, wa_ref[...])
        b_ref[...] = _dot(h, wb_ref[...])

    return pl.pallas_call(
        body,
        grid=(GN,),
        in_specs=[
            pl.BlockSpec((TN, EMB_ATOM), lambda i: (i, 0)),
            pl.BlockSpec((EMB_ATOM, EMB_EDGE), lambda i: (0, 0)),
            pl.BlockSpec((EMB_ATOM, EMB_EDGE), lambda i: (0, 0)),
        ],
        out_specs=[pl.BlockSpec((TN, EMB_EDGE), lambda i: (i, 0))] * 2,
        out_shape=[jax.ShapeDtypeStruct((NP, EMB_EDGE), jnp.float32)] * 2,
    )(h0, wa, wb)


def _tc_edge_init(gpos, asrc, bdst, w_rbf, wc):
    gap = CUTOFF / (NUM_RADIAL - 1)
    coeff = -0.5 / (gap * gap)

    def body(ps_ref, pd_ref, a_ref, b_ref, wr_ref, wc_ref, m_ref, r_ref, u_ref):
        diff = pd_ref[...] - ps_ref[...]
        d2 = jnp.sum(diff * diff, axis=1, keepdims=True)
        dd = jnp.sqrt(d2 + 1e-12)
        u_ref[...] = (diff / dd)[:, :8]
        offs = lax.broadcasted_iota(jnp.int32, (TE, NUM_RADIAL), 1).astype(jnp.float32) * gap
        g = jnp.exp(coeff * (dd - offs) ** 2)
        x = dd / CUTOFF
        x2 = x * x
        x5 = x2 * x2 * x
        env = 1.0 - 21.0 * x5 + 35.0 * (x5 * x) - 15.0 * (x5 * x2)
        env = jnp.where(x < 1.0, env, 0.0)
        rh = _dot(g * env, wr_ref[...])
        r_ref[...] = rh
        m_ref[...] = _silu(a_ref[...] + b_ref[...] + _dot(rh, wc_ref[...]))

    return pl.pallas_call(
        body,
        grid=(GE,),
        in_specs=[
            pl.BlockSpec((TE, 16), lambda i: (i, 0)),
            pl.BlockSpec((TE, 16), lambda i: (i + GE, 0)),
            pl.BlockSpec((TE, EMB_EDGE), lambda i: (i, 0)),
            pl.BlockSpec((TE, EMB_EDGE), lambda i: (i, 0)),
            pl.BlockSpec((NUM_RADIAL, EMB_RBF), lambda i: (0, 0)),
            pl.BlockSpec((EMB_RBF, EMB_EDGE), lambda i: (0, 0)),
        ],
        out_specs=[
            pl.BlockSpec((TE, EMB_EDGE), lambda i: (i, 0)),
            pl.BlockSpec((TE, EMB_RBF), lambda i: (i, 0)),
            pl.BlockSpec((TE, 8), lambda i: (i, 0)),
        ],
        out_shape=[
            jax.ShapeDtypeStruct((EP, EMB_EDGE), jnp.float32),
            jax.ShapeDtypeStruct((EP, EMB_RBF), jnp.float32),
            jax.ShapeDtypeStruct((EP, 8), jnp.float32),
        ],
    )(gpos, gpos, asrc, bdst, w_rbf, wc)


def _tc_edge_block(m_prev, rbfh, w1b, wsb, g):
    has_g = g is not None

    def body(*refs):
        if has_g:
            m_ref, r_ref, gs_ref, gd_ref, w1_ref, ws_ref, mo_ref, m2_ref = refs
            m = m_ref[...] + _silu(gs_ref[...] + gd_ref[...])
        else:
            m_ref, r_ref, w1_ref, ws_ref, mo_ref, m2_ref = refs
            m = m_ref[...]
        m2 = _silu(_dot(m, w1_ref[...])) * _dot(r_ref[...], ws_ref[...])
        m2_ref[...] = m2
        mo_ref[...] = m + m2

    in_specs = [
        pl.BlockSpec((TE, EMB_EDGE), lambda i: (i, 0)),
        pl.BlockSpec((TE, EMB_RBF), lambda i: (i, 0)),
    ]
    args = [m_prev, rbfh]
    if has_g:
        in_specs += [
            pl.BlockSpec((TE, EMB_EDGE), lambda i: (i, 0)),
            pl.BlockSpec((TE, EMB_EDGE), lambda i: (i + GE, 0)),
        ]
        args += [g, g]
    in_specs += [
        pl.BlockSpec((EMB_EDGE, EMB_EDGE), lambda i: (0, 0)),
        pl.BlockSpec((EMB_RBF, EMB_EDGE), lambda i: (0, 0)),
    ]
    args += [w1b, wsb]
    return pl.pallas_call(
        body,
        grid=(GE,),
        in_specs=in_specs,
        out_specs=[pl.BlockSpec((TE, EMB_EDGE), lambda i: (i, 0))] * 2,
        out_shape=[jax.ShapeDtypeStruct((EP, EMB_EDGE), jnp.float32)] * 2,
    )(*args)


def _tc_atom_update(s_part, h, w2b, w3b):
    def body(s0_ref, s1_ref, h_ref, w2_ref, w3_ref, h_ref_o, p_ref):
        agg = s0_ref[0] + s1_ref[0]
        hn = h_ref[...] + _silu(_dot(agg, w2_ref[...]))
        h_ref_o[...] = hn
        p_ref[...] = _dot(hn, w3_ref[...])

    return pl.pallas_call(
        body,
        grid=(GN,),
        in_specs=[
            pl.BlockSpec((1, TN, EMB_EDGE), lambda i: (0, i, 0)),
            pl.BlockSpec((1, TN, EMB_EDGE), lambda i: (1, i, 0)),
            pl.BlockSpec((TN, EMB_ATOM), lambda i: (i, 0)),
            pl.BlockSpec((EMB_EDGE, EMB_ATOM), lambda i: (0, 0)),
            pl.BlockSpec((EMB_ATOM, EMB_EDGE), lambda i: (0, 0)),
        ],
        out_specs=[
            pl.BlockSpec((TN, EMB_ATOM), lambda i: (i, 0)),
            pl.BlockSpec((TN, EMB_EDGE), lambda i: (i, 0)),
        ],
        out_shape=[
            jax.ShapeDtypeStruct((NP, EMB_ATOM), jnp.float32),
            jax.ShapeDtypeStruct((NP, EMB_EDGE), jnp.float32),
        ],
    )(s_part, s_part, h, w2b, w3b)


def _tc_force_edge(m3, g, u8, w_f):
    def body(m_ref, gs_ref, gd_ref, u_ref, wf_ref, f_ref):
        m = m_ref[...] + _silu(gs_ref[...] + gd_ref[...])
        f = _dot(m, wf_ref[...])
        f_ref[...] = f * u_ref[...]

    return pl.pallas_call(
        body,
        grid=(GE,),
        in_specs=[
            pl.BlockSpec((TE, EMB_EDGE), lambda i: (i, 0)),
            pl.BlockSpec((TE, EMB_EDGE), lambda i: (i, 0)),
            pl.BlockSpec((TE, EMB_EDGE), lambda i: (i + GE, 0)),
            pl.BlockSpec((TE, 8), lambda i: (i, 0)),
            pl.BlockSpec((EMB_EDGE, 1), lambda i: (0, 0)),
        ],
        out_specs=[pl.BlockSpec((TE, 8), lambda i: (i, 0))],
        out_shape=[jax.ShapeDtypeStruct((EP, 8), jnp.float32)],
    )(m3, g, g, u8, w_f)[0]


def _tc_out(h, sf, w_e):
    def body(h_ref, f0_ref, f1_ref, we_ref, o_ref):
        e = _dot(h_ref[...], we_ref[...])
        f = f0_ref[0] + f1_ref[0]
        o_ref[:, 0:1] = e
        o_ref[:, 1:4] = f[:, 0:3]

    return pl.pallas_call(
        body,
        grid=(GN,),
        in_specs=[
            pl.BlockSpec((TN, EMB_ATOM), lambda i: (i, 0)),
            pl.BlockSpec((1, TN, 8), lambda i: (0, i, 0)),
            pl.BlockSpec((1, TN, 8), lambda i: (1, i, 0)),
            pl.BlockSpec((EMB_ATOM, 1), lambda i: (0, 0)),
        ],
        out_specs=[pl.BlockSpec((TN, 4), lambda i: (i, 0))],
        out_shape=[jax.ShapeDtypeStruct((NP, 4), jnp.float32)],
    )(h, sf, sf, w_e)[0]


def kernel(atomic_numbers, pos, edge_index, atom_emb, W_rbf, W_edge_in, W1, Ws, W2, W3, W_E, W_F):
    src = edge_index[0].astype(jnp.int32)
    dst = edge_index[1].astype(jnp.int32)
    pad_e = jnp.full((EP - E,), N, jnp.int32)
    srcp = jnp.concatenate([src, pad_e])
    dstp = jnp.concatenate([dst, pad_e])
    idx2 = jnp.concatenate([srcp, dstp])
    anp = jnp.concatenate(
        [atomic_numbers.astype(jnp.int32), jnp.zeros((NP - N,), jnp.int32)])
    pos16 = jnp.zeros((NP, 16), jnp.float32).at[:N, :3].set(pos)
    zeros32 = jnp.zeros((NP, EMB_EDGE), jnp.float32)
    zeros8 = jnp.zeros((NP, 8), jnp.float32)
    wa = W_edge_in[:EMB_ATOM]
    wb = W_edge_in[EMB_ATOM:2 * EMB_ATOM]
    wc = W_edge_in[2 * EMB_ATOM:]

    h = _sc_gather(atom_emb, anp, EMB_ATOM)
    a0, b0 = _tc_atom_init(h, wa, wb)
    asrc = _sc_gather(a0, srcp, EMB_EDGE)
    bdst = _sc_gather(b0, dstp, EMB_EDGE)
    gpos = _sc_gather(pos16, idx2, 16)
    m, rbfh, u8 = _tc_edge_init(gpos, asrc, bdst, W_rbf, wc)

    g = None
    for blk in range(NUM_BLOCKS):
        m, m2 = _tc_edge_block(m, rbfh, W1[blk], Ws[blk], g)
        s_part = _sc_scatter_add(m2, dstp, zeros32, EMB_EDGE)
        h, p = _tc_atom_update(s_part, h, W2[blk], W3[blk])
        g = _sc_gather(p, idx2, EMB_EDGE)

    fe = _tc_force_edge(m, g, u8, W_F)
    sf = _sc_scatter_add(fe, dstp, zeros8, 8)
    out = _tc_out(h, sf, W_E)
    return out[:N]


# speedup vs baseline: 2.2385x; 1.1425x over previous
import functools

import jax
import jax.numpy as jnp
from jax import lax
from jax.experimental import pallas as pl
from jax.experimental.pallas import tpu as pltpu
from jax.experimental.pallas import tpu_sc as plsc

N = 50000
E = 800000
NUM_RADIAL = 64
EMB_RBF = 16
EMB_ATOM = 64
EMB_EDGE = 32
NUM_BLOCKS = 4
CUTOFF = 6.0

NC = 2
NS = 16
NW = NC * NS
CH = 128

NP = 53248
EP = 802816
TE = 4096
TN = 4096
GE = EP // TE
GN = NP // TN

def _silu(x):
    return x / (1.0 + jnp.exp(-x))


U = 4


def _sc_gather(table, idx, d):
    b = idx.shape[0]
    per_w = b // NW
    n_chunks = per_w // CH
    assert n_chunks % U == 0
    nk = n_chunks // U

    @functools.partial(
        pl.kernel,
        out_type=jax.ShapeDtypeStruct((b, d), jnp.float32),
        mesh=plsc.VectorSubcoreMesh(core_axis_name="c", subcore_axis_name="s"),
        compiler_params=pltpu.CompilerParams(use_tc_tiling_on_sc=False),
        scratch_types=[
            pltpu.VMEM((U, CH), jnp.int32),
            pltpu.VMEM((U, CH, d), jnp.float32),
            [pltpu.SemaphoreType.DMA] * U,
            [pltpu.SemaphoreType.DMA] * U,
            [pltpu.SemaphoreType.DMA] * U,
        ],
    )
    def k(table_h, idx_h, out_h, idx_v, rows_v, si, sg, sw):
        w = lax.axis_index("s") * NC + lax.axis_index("c")
        base = w * per_w
        for bq in range(U):
            pltpu.async_copy(idx_h.at[pl.ds(base + bq * CH, CH)],
                             idx_v.at[bq], si[bq])

        def outer(kk, carry):
            for bq in range(U):
                off = base + (kk * U + bq) * CH
                pltpu.make_async_copy(
                    idx_h.at[pl.ds(off, CH)], idx_v.at[bq], si[bq]).wait()

                @pl.when(kk > 0)
                def _wait_w():
                    pltpu.make_async_copy(
                        rows_v.at[bq], out_h.at[pl.ds(off, CH)], sw[bq]).wait()

                pltpu.async_copy(table_h.at[idx_v.at[bq]], rows_v.at[bq],
                                 sg[bq]).wait()
                pltpu.async_copy(rows_v.at[bq], out_h.at[pl.ds(off, CH)],
                                 sw[bq])

                @pl.when(kk < nk - 1)
                def _next_i():
                    pltpu.async_copy(idx_h.at[pl.ds(off + U * CH, CH)],
                                     idx_v.at[bq], si[bq])
            return carry

        lax.fori_loop(0, nk, outer, 0)
        for bq in range(U):
            pltpu.make_async_copy(
                rows_v.at[bq], out_h.at[pl.ds(base, CH)], sw[bq]).wait()

    return k(table, idx)


def _sc_scatter_add(vals, idx, zeros, d):
    half = EP // NC
    per_t = half // NS
    n_chunks = per_t // CH
    rows_t = NP // NS

    assert n_chunks % U == 0
    nk = n_chunks // U

    def k(vals_h, idx_h, zeros_h, out_h, idx_v, vals_v, acc, si, sv, sa):
        c = lax.axis_index("c")
        s = lax.axis_index("s")
        r0 = s * rows_t
        pltpu.sync_copy(zeros_h.at[pl.ds(r0, rows_t)], acc.at[pl.ds(r0, rows_t)])
        plsc.subcore_barrier()
        base = c * half + s * per_t
        for bq in range(U):
            pltpu.async_copy(idx_h.at[pl.ds(base + bq * CH, CH)],
                             idx_v.at[bq], si[bq])
            pltpu.async_copy(vals_h.at[pl.ds(base + bq * CH, CH)],
                             vals_v.at[bq], sv[bq])

        def outer(kk, carry):
            for bq in range(U):
                off = base + (kk * U + bq) * CH
                pltpu.make_async_copy(
                    idx_h.at[pl.ds(off, CH)], idx_v.at[bq], si[bq]).wait()
                pltpu.make_async_copy(
                    vals_h.at[pl.ds(off, CH)], vals_v.at[bq], sv[bq]).wait()
                pltpu.async_copy(vals_v.at[bq], acc.at[idx_v.at[bq]],
                                 sa[bq], add=True).wait()

                @pl.when(kk < nk - 1)
                def _next():
                    pltpu.async_copy(idx_h.at[pl.ds(off + U * CH, CH)],
                                     idx_v.at[bq], si[bq])
                    pltpu.async_copy(vals_h.at[pl.ds(off + U * CH, CH)],
                                     vals_v.at[bq], sv[bq])
            return carry

        lax.fori_loop(0, nk, outer, 0)
        plsc.subcore_barrier()
        pltpu.sync_copy(acc.at[pl.ds(r0, rows_t)], out_h.at[c, pl.ds(r0, rows_t)])

    kk = functools.partial(
        pl.kernel,
        out_type=jax.ShapeDtypeStruct((NC, NP, d), jnp.float32),
        mesh=plsc.VectorSubcoreMesh(core_axis_name="c", subcore_axis_name="s"),
        compiler_params=pltpu.CompilerParams(use_tc_tiling_on_sc=False),
        scratch_types=[
            pltpu.VMEM((U, CH), jnp.int32),
            pltpu.VMEM((U, CH, d), jnp.float32),
            pltpu.VMEM_SHARED((NP, d), jnp.float32),
            [pltpu.SemaphoreType.DMA] * U,
            [pltpu.SemaphoreType.DMA] * U,
            [pltpu.SemaphoreType.DMA] * U,
        ],
    )(k)
    return kk(vals, idx, zeros)


def _dot(a, b):
    return jnp.dot(a, b, preferred_element_type=jnp.float32,
                   precision=lax.Precision.HIGHEST)


def _tc_atom_init(h0, wa, wb):
    def body(h_ref, wa_ref, wb_ref, a_ref, b_ref):
        h = h_ref[...]
        a_ref[...] = _dot(h, wa_ref[...])
        b_ref[...] = _dot(h, wb_ref[...])

    return pl.pallas_call(
        body,
        grid=(GN,),
        in_specs=[
            pl.BlockSpec((TN, EMB_ATOM), lambda i: (i, 0)),
            pl.BlockSpec((EMB_ATOM, EMB_EDGE), lambda i: (0, 0)),
            pl.BlockSpec((EMB_ATOM, EMB_EDGE), lambda i: (0, 0)),
        ],
        out_specs=[pl.BlockSpec((TN, EMB_EDGE), lambda i: (i, 0))] * 2,
        out_shape=[jax.ShapeDtypeStruct((NP, EMB_EDGE), jnp.float32)] * 2,
    )(h0, wa, wb)


def _tc_edge_init(gpos, asrc, bdst, w_rbf, wc):
    gap = CUTOFF / (NUM_RADIAL - 1)
    coeff = -0.5 / (gap * gap)

    def body(ps_ref, pd_ref, a_ref, b_ref, wr_ref, wc_ref, m_ref, r_ref, u_ref):
        diff = pd_ref[...] - ps_ref[...]
        d2 = jnp.sum(diff * diff, axis=1, keepdims=True)
        dd = jnp.sqrt(d2 + 1e-12)
        u_ref[...] = (diff / dd)[:, :8]
        offs = lax.broadcasted_iota(jnp.int32, (TE, NUM_RADIAL), 1).astype(jnp.float32) * gap
        g = jnp.exp(coeff * (dd - offs) ** 2)
        x = dd / CUTOFF
        x2 = x * x
        x5 = x2 * x2 * x
        env = 1.0 - 21.0 * x5 + 35.0 * (x5 * x) - 15.0 * (x5 * x2)
        env = jnp.where(x < 1.0, env, 0.0)
        rh = _dot(g * env, wr_ref[...])
        r_ref[...] = rh
        m_ref[...] = _silu(a_ref[...] + b_ref[...] + _dot(rh, wc_ref[...]))

    return pl.pallas_call(
        body,
        grid=(GE,),
        in_specs=[
            pl.BlockSpec((TE, 16), lambda i: (i, 0)),
            pl.BlockSpec((TE, 16), lambda i: (i + GE, 0)),
            pl.BlockSpec((TE, EMB_EDGE), lambda i: (i, 0)),
            pl.BlockSpec((TE, EMB_EDGE), lambda i: (i, 0)),
            pl.BlockSpec((NUM_RADIAL, EMB_RBF), lambda i: (0, 0)),
            pl.BlockSpec((EMB_RBF, EMB_EDGE), lambda i: (0, 0)),
        ],
        out_specs=[
            pl.BlockSpec((TE, EMB_EDGE), lambda i: (i, 0)),
            pl.BlockSpec((TE, EMB_RBF), lambda i: (i, 0)),
            pl.BlockSpec((TE, 8), lambda i: (i, 0)),
        ],
        out_shape=[
            jax.ShapeDtypeStruct((EP, EMB_EDGE), jnp.float32),
            jax.ShapeDtypeStruct((EP, EMB_RBF), jnp.float32),
            jax.ShapeDtypeStruct((EP, 8), jnp.float32),
        ],
    )(gpos, gpos, asrc, bdst, w_rbf, wc)


def _tc_edge_block(m_prev, rbfh, w1b, wsb, g):
    has_g = g is not None

    def body(*refs):
        if has_g:
            m_ref, r_ref, gs_ref, gd_ref, w1_ref, ws_ref, mo_ref, m2_ref = refs
            m = m_ref[...] + _silu(gs_ref[...] + gd_ref[...])
        else:
            m_ref, r_ref, w1_ref, ws_ref, mo_ref, m2_ref = refs
            m = m_ref[...]
        m2 = _silu(_dot(m, w1_ref[...])) * _dot(r_ref[...], ws_ref[...])
        m2_ref[...] = m2
        mo_ref[...] = m + m2

    in_specs = [
        pl.BlockSpec((TE, EMB_EDGE), lambda i: (i, 0)),
        pl.BlockSpec((TE, EMB_RBF), lambda i: (i, 0)),
    ]
    args = [m_prev, rbfh]
    if has_g:
        in_specs += [
            pl.BlockSpec((TE, EMB_EDGE), lambda i: (i, 0)),
            pl.BlockSpec((TE, EMB_EDGE), lambda i: (i + GE, 0)),
        ]
        args += [g, g]
    in_specs += [
        pl.BlockSpec((EMB_EDGE, EMB_EDGE), lambda i: (0, 0)),
        pl.BlockSpec((EMB_RBF, EMB_EDGE), lambda i: (0, 0)),
    ]
    args += [w1b, wsb]
    return pl.pallas_call(
        body,
        grid=(GE,),
        in_specs=in_specs,
        out_specs=[pl.BlockSpec((TE, EMB_EDGE), lambda i: (i, 0))] * 2,
        out_shape=[jax.ShapeDtypeStruct((EP, EMB_EDGE), jnp.float32)] * 2,
    )(*args)


def _tc_atom_update(s_part, h, w2b, w3b):
    def body(s0_ref, s1_ref, h_ref, w2_ref, w3_ref, h_ref_o, p_ref):
        agg = s0_ref[0] + s1_ref[0]
        hn = h_ref[...] + _silu(_dot(agg, w2_ref[...]))
        h_ref_o[...] = hn
        p_ref[...] = _dot(hn, w3_ref[...])

    return pl.pallas_call(
        body,
        grid=(GN,),
        in_specs=[
            pl.BlockSpec((1, TN, EMB_EDGE), lambda i: (0, i, 0)),
            pl.BlockSpec((1, TN, EMB_EDGE), lambda i: (1, i, 0)),
            pl.BlockSpec((TN, EMB_ATOM), lambda i: (i, 0)),
            pl.BlockSpec((EMB_EDGE, EMB_ATOM), lambda i: (0, 0)),
            pl.BlockSpec((EMB_ATOM, EMB_EDGE), lambda i: (0, 0)),
        ],
        out_specs=[
            pl.BlockSpec((TN, EMB_ATOM), lambda i: (i, 0)),
            pl.BlockSpec((TN, EMB_EDGE), lambda i: (i, 0)),
        ],
        out_shape=[
            jax.ShapeDtypeStruct((NP, EMB_ATOM), jnp.float32),
            jax.ShapeDtypeStruct((NP, EMB_EDGE), jnp.float32),
        ],
    )(s_part, s_part, h, w2b, w3b)


def _tc_force_edge(m3, g, u8, w_f):
    def body(m_ref, gs_ref, gd_ref, u_ref, wf_ref, f_ref):
        m = m_ref[...] + _silu(gs_ref[...] + gd_ref[...])
        f = _dot(m, wf_ref[...])
        f_ref[...] = f * u_ref[...]

    return pl.pallas_call(
        body,
        grid=(GE,),
        in_specs=[
            pl.BlockSpec((TE, EMB_EDGE), lambda i: (i, 0)),
            pl.BlockSpec((TE, EMB_EDGE), lambda i: (i, 0)),
            pl.BlockSpec((TE, EMB_EDGE), lambda i: (i + GE, 0)),
            pl.BlockSpec((TE, 8), lambda i: (i, 0)),
            pl.BlockSpec((EMB_EDGE, 1), lambda i: (0, 0)),
        ],
        out_specs=[pl.BlockSpec((TE, 8), lambda i: (i, 0))],
        out_shape=[jax.ShapeDtypeStruct((EP, 8), jnp.float32)],
    )(m3, g, g, u8, w_f)[0]


def _tc_out(h, sf, w_e):
    def body(h_ref, f0_ref, f1_ref, we_ref, o_ref):
        e = _dot(h_ref[...], we_ref[...])
        f = f0_ref[0] + f1_ref[0]
        o_ref[:, 0:1] = e
        o_ref[:, 1:4] = f[:, 0:3]

    return pl.pallas_call(
        body,
        grid=(GN,),
        in_specs=[
            pl.BlockSpec((TN, EMB_ATOM), lambda i: (i, 0)),
            pl.BlockSpec((1, TN, 8), lambda i: (0, i, 0)),
            pl.BlockSpec((1, TN, 8), lambda i: (1, i, 0)),
            pl.BlockSpec((EMB_ATOM, 1), lambda i: (0, 0)),
        ],
        out_specs=[pl.BlockSpec((TN, 4), lambda i: (i, 0))],
        out_shape=[jax.ShapeDtypeStruct((NP, 4), jnp.float32)],
    )(h, sf, sf, w_e)[0]


def kernel(atomic_numbers, pos, edge_index, atom_emb, W_rbf, W_edge_in, W1, Ws, W2, W3, W_E, W_F):
    src = edge_index[0].astype(jnp.int32)
    dst = edge_index[1].astype(jnp.int32)
    pad_e = jnp.full((EP - E,), N, jnp.int32)
    srcp = jnp.concatenate([src, pad_e])
    dstp = jnp.concatenate([dst, pad_e])
    idx2 = jnp.concatenate([srcp, dstp])
    anp = jnp.concatenate(
        [atomic_numbers.astype(jnp.int32), jnp.zeros((65536 - N,), jnp.int32)])
    pos16 = jnp.zeros((NP, 16), jnp.float32).at[:N, :3].set(pos)
    zeros32 = jnp.zeros((NP, EMB_EDGE), jnp.float32)
    zeros8 = jnp.zeros((NP, 8), jnp.float32)
    wa = W_edge_in[:EMB_ATOM]
    wb = W_edge_in[EMB_ATOM:2 * EMB_ATOM]
    wc = W_edge_in[2 * EMB_ATOM:]

    h = _sc_gather(atom_emb, anp, EMB_ATOM)
    a0, b0 = _tc_atom_init(h, wa, wb)
    asrc = _sc_gather(a0, srcp, EMB_EDGE)
    bdst = _sc_gather(b0, dstp, EMB_EDGE)
    gpos = _sc_gather(pos16, idx2, 16)
    m, rbfh, u8 = _tc_edge_init(gpos, asrc, bdst, W_rbf, wc)

    g = None
    for blk in range(NUM_BLOCKS):
        m, m2 = _tc_edge_block(m, rbfh, W1[blk], Ws[blk], g)
        s_part = _sc_scatter_add(m2, dstp, zeros32, EMB_EDGE)
        h, p = _tc_atom_update(s_part, h, W2[blk], W3[blk])
        g = _sc_gather(p, idx2, EMB_EDGE)

    fe = _tc_force_edge(m, g, u8, W_F)
    sf = _sc_scatter_add(fe, dstp, zeros8, 8)
    out = _tc_out(h, sf, W_E)
    return out[:N]


# speedup vs baseline: 2.2504x; 1.0053x over previous
import functools

import jax
import jax.numpy as jnp
from jax import lax
from jax.experimental import pallas as pl
from jax.experimental.pallas import tpu as pltpu
from jax.experimental.pallas import tpu_sc as plsc

N = 50000
E = 800000
NUM_RADIAL = 64
EMB_RBF = 16
EMB_ATOM = 64
EMB_EDGE = 32
NUM_BLOCKS = 4
CUTOFF = 6.0

NC = 2
NS = 16
NW = NC * NS
CH = 128

NP = 53248
EP = 802816
TE = 4096
TN = 4096
GE = EP // TE
GN = NP // TN

def _silu(x):
    return x / (1.0 + jnp.exp(-x))


U = 4


def _sc_gather(table, idx, d):
    b = idx.shape[0]
    per_w = b // NW
    n_chunks = per_w // CH
    assert n_chunks % U == 0
    nk = n_chunks // U
    lag = U - 1

    @functools.partial(
        pl.kernel,
        out_type=jax.ShapeDtypeStruct((b, d), jnp.float32),
        mesh=plsc.VectorSubcoreMesh(core_axis_name="c", subcore_axis_name="s"),
        compiler_params=pltpu.CompilerParams(use_tc_tiling_on_sc=False),
        scratch_types=[
            pltpu.VMEM((U, CH), jnp.int32),
            pltpu.VMEM((U, CH, d), jnp.float32),
            [pltpu.SemaphoreType.DMA] * U,
            [pltpu.SemaphoreType.DMA] * U,
            [pltpu.SemaphoreType.DMA] * U,
        ],
    )
    def k(table_h, idx_h, out_h, idx_v, rows_v, si, sg, sw):
        w = lax.axis_index("s") * NC + lax.axis_index("c")
        base = w * per_w
        for bq in range(U):
            pltpu.async_copy(idx_h.at[pl.ds(base + bq * CH, CH)],
                             idx_v.at[bq], si[bq])

        def launch(i, bq):
            pltpu.make_async_copy(
                idx_h.at[pl.ds(base + i * CH, CH)], idx_v.at[bq], si[bq]).wait()
            pltpu.async_copy(table_h.at[idx_v.at[bq]], rows_v.at[bq], sg[bq])

        def retire(j, bp):
            off = base + j * CH
            pltpu.make_async_copy(
                table_h.at[idx_v.at[bp]], rows_v.at[bp], sg[bp]).wait()
            pltpu.async_copy(rows_v.at[bp], out_h.at[pl.ds(off, CH)], sw[bp])

        def outer(kk, carry):
            for bq in range(U):
                i = kk * U + bq

                @pl.when(i >= U)
                def _wait_w():
                    pltpu.make_async_copy(
                        rows_v.at[bq], out_h.at[pl.ds(base, CH)], sw[bq]).wait()

                launch(i, bq)
                bp = (bq + 1) % U
                j = i - lag

                @pl.when(j >= 0)
                def _retire():
                    retire(j, bp)

                @pl.when((j >= 0) & (j + U < n_chunks))
                def _prefetch():
                    pltpu.async_copy(
                        idx_h.at[pl.ds(base + (j + U) * CH, CH)],
                        idx_v.at[bp], si[bp])
            return carry

        lax.fori_loop(0, nk, outer, 0)
        for t in range(lag):
            j = n_chunks - lag + t
            retire(j, j % U)
        for bq in range(U):
            pltpu.make_async_copy(
                rows_v.at[bq], out_h.at[pl.ds(base, CH)], sw[bq]).wait()

    return k(table, idx)


def _sc_scatter_add(vals, idx, zeros, d):
    half = EP // NC
    per_t = half // NS
    n_chunks = per_t // CH
    rows_t = NP // NS

    assert n_chunks % U == 0
    nk = n_chunks // U

    def k(vals_h, idx_h, zeros_h, out_h, idx_v, vals_v, acc, si, sv, sa):
        c = lax.axis_index("c")
        s = lax.axis_index("s")
        r0 = s * rows_t
        pltpu.sync_copy(zeros_h.at[pl.ds(r0, rows_t)], acc.at[pl.ds(r0, rows_t)])
        plsc.subcore_barrier()
        base = c * half + s * per_t
        for bq in range(U):
            pltpu.async_copy(idx_h.at[pl.ds(base + bq * CH, CH)],
                             idx_v.at[bq], si[bq])
            pltpu.async_copy(vals_h.at[pl.ds(base + bq * CH, CH)],
                             vals_v.at[bq], sv[bq])

        def launch(i, bq):
            off = base + i * CH
            pltpu.make_async_copy(
                idx_h.at[pl.ds(off, CH)], idx_v.at[bq], si[bq]).wait()
            pltpu.make_async_copy(
                vals_h.at[pl.ds(off, CH)], vals_v.at[bq], sv[bq]).wait()
            pltpu.async_copy(vals_v.at[bq], acc.at[idx_v.at[bq]],
                             sa[bq], add=True)

        def retire(j, bp):
            pltpu.make_async_copy(vals_h.at[pl.ds(base, CH)], vals_v.at[bp],
                                  sa[bp]).wait()

            @pl.when(j + U < n_chunks)
            def _prefetch():
                off = base + (j + U) * CH
                pltpu.async_copy(idx_h.at[pl.ds(off, CH)], idx_v.at[bp], si[bp])
                pltpu.async_copy(vals_h.at[pl.ds(off, CH)], vals_v.at[bp], sv[bp])

        lag = U - 1

        def outer(kk, carry):
            for bq in range(U):
                i = kk * U + bq
                launch(i, bq)
                bp = (bq + 1) % U
                j = i - lag

                @pl.when(j >= 0)
                def _retire():
                    retire(j, bp)
            return carry

        lax.fori_loop(0, nk, outer, 0)
        for t in range(lag):
            j = n_chunks - lag + t
            retire(j, j % U)
        plsc.subcore_barrier()
        pltpu.sync_copy(acc.at[pl.ds(r0, rows_t)], out_h.at[c, pl.ds(r0, rows_t)])

    kk = functools.partial(
        pl.kernel,
        out_type=jax.ShapeDtypeStruct((NC, NP, d), jnp.float32),
        mesh=plsc.VectorSubcoreMesh(core_axis_name="c", subcore_axis_name="s"),
        compiler_params=pltpu.CompilerParams(use_tc_tiling_on_sc=False),
        scratch_types=[
            pltpu.VMEM((U, CH), jnp.int32),
            pltpu.VMEM((U, CH, d), jnp.float32),
            pltpu.VMEM_SHARED((NP, d), jnp.float32),
            [pltpu.SemaphoreType.DMA] * U,
            [pltpu.SemaphoreType.DMA] * U,
            [pltpu.SemaphoreType.DMA] * U,
        ],
    )(k)
    return kk(vals, idx, zeros)


def _dot(a, b):
    return jnp.dot(a, b, preferred_element_type=jnp.float32,
                   precision=lax.Precision.HIGHEST)


def _tc_atom_init(h0, wa, wb):
    def body(h_ref, wa_ref, wb_ref, a_ref, b_ref):
        h = h_ref[...]
        a_ref[...] = _dot(h, wa_ref[...])
        b_ref[...] = _dot(h, wb_ref[...])

    return pl.pallas_call(
        body,
        grid=(GN,),
        in_specs=[
            pl.BlockSpec((TN, EMB_ATOM), lambda i: (i, 0)),
            pl.BlockSpec((EMB_ATOM, EMB_EDGE), lambda i: (0, 0)),
            pl.BlockSpec((EMB_ATOM, EMB_EDGE), lambda i: (0, 0)),
        ],
        out_specs=[pl.BlockSpec((TN, EMB_EDGE), lambda i: (i, 0))] * 2,
        out_shape=[jax.ShapeDtypeStruct((NP, EMB_EDGE), jnp.float32)] * 2,
    )(h0, wa, wb)


def _tc_edge_init(gpos, asrc, bdst, w_rbf, wc):
    gap = CUTOFF / (NUM_RADIAL - 1)
    coeff = -0.5 / (gap * gap)

    def body(ps_ref, pd_ref, a_ref, b_ref, wr_ref, wc_ref, m_ref, r_ref, u_ref):
        diff = pd_ref[...] - ps_ref[...]
        d2 = jnp.sum(diff * diff, axis=1, keepdims=True)
        dd = jnp.sqrt(d2 + 1e-12)
        u_ref[...] = (diff / dd)[:, :8]
        offs = lax.broadcasted_iota(jnp.int32, (TE, NUM_RADIAL), 1).astype(jnp.float32) * gap
        g = jnp.exp(coeff * (dd - offs) ** 2)
        x = dd / CUTOFF
        x2 = x * x
        x5 = x2 * x2 * x
        env = 1.0 - 21.0 * x5 + 35.0 * (x5 * x) - 15.0 * (x5 * x2)
        env = jnp.where(x < 1.0, env, 0.0)
        rh = _dot(g * env, wr_ref[...])
        r_ref[...] = rh
        m_ref[...] = _silu(a_ref[...] + b_ref[...] + _dot(rh, wc_ref[...]))

    return pl.pallas_call(
        body,
        grid=(GE,),
        in_specs=[
            pl.BlockSpec((TE, 16), lambda i: (i, 0)),
            pl.BlockSpec((TE, 16), lambda i: (i + GE, 0)),
            pl.BlockSpec((TE, EMB_EDGE), lambda i: (i, 0)),
            pl.BlockSpec((TE, EMB_EDGE), lambda i: (i, 0)),
            pl.BlockSpec((NUM_RADIAL, EMB_RBF), lambda i: (0, 0)),
            pl.BlockSpec((EMB_RBF, EMB_EDGE), lambda i: (0, 0)),
        ],
        out_specs=[
            pl.BlockSpec((TE, EMB_EDGE), lambda i: (i, 0)),
            pl.BlockSpec((TE, EMB_RBF), lambda i: (i, 0)),
            pl.BlockSpec((TE, 8), lambda i: (i, 0)),
        ],
        out_shape=[
            jax.ShapeDtypeStruct((EP, EMB_EDGE), jnp.float32),
            jax.ShapeDtypeStruct((EP, EMB_RBF), jnp.float32),
            jax.ShapeDtypeStruct((EP, 8), jnp.float32),
        ],
    )(gpos, gpos, asrc, bdst, w_rbf, wc)


def _tc_edge_block(m_prev, rbfh, w1b, wsb, g):
    has_g = g is not None

    def body(*refs):
        if has_g:
            m_ref, r_ref, gs_ref, gd_ref, w1_ref, ws_ref, mo_ref, m2_ref = refs
            m = m_ref[...] + _silu(gs_ref[...] + gd_ref[...])
        else:
            m_ref, r_ref, w1_ref, ws_ref, mo_ref, m2_ref = refs
            m = m_ref[...]
        m2 = _silu(_dot(m, w1_ref[...])) * _dot(r_ref[...], ws_ref[...])
        m2_ref[...] = m2
        mo_ref[...] = m + m2

    in_specs = [
        pl.BlockSpec((TE, EMB_EDGE), lambda i: (i, 0)),
        pl.BlockSpec((TE, EMB_RBF), lambda i: (i, 0)),
    ]
    args = [m_prev, rbfh]
    if has_g:
        in_specs += [
            pl.BlockSpec((TE, EMB_EDGE), lambda i: (i, 0)),
            pl.BlockSpec((TE, EMB_EDGE), lambda i: (i + GE, 0)),
        ]
        args += [g, g]
    in_specs += [
        pl.BlockSpec((EMB_EDGE, EMB_EDGE), lambda i: (0, 0)),
        pl.BlockSpec((EMB_RBF, EMB_EDGE), lambda i: (0, 0)),
    ]
    args += [w1b, wsb]
    return pl.pallas_call(
        body,
        grid=(GE,),
        in_specs=in_specs,
        out_specs=[pl.BlockSpec((TE, EMB_EDGE), lambda i: (i, 0))] * 2,
        out_shape=[jax.ShapeDtypeStruct((EP, EMB_EDGE), jnp.float32)] * 2,
    )(*args)


def _tc_atom_update(s_part, h, w2b, w3b):
    def body(s0_ref, s1_ref, h_ref, w2_ref, w3_ref, h_ref_o, p_ref):
        agg = s0_ref[0] + s1_ref[0]
        hn = h_ref[...] + _silu(_dot(agg, w2_ref[...]))
        h_ref_o[...] = hn
        p_ref[...] = _dot(hn, w3_ref[...])

    return pl.pallas_call(
        body,
        grid=(GN,),
        in_specs=[
            pl.BlockSpec((1, TN, EMB_EDGE), lambda i: (0, i, 0)),
            pl.BlockSpec((1, TN, EMB_EDGE), lambda i: (1, i, 0)),
            pl.BlockSpec((TN, EMB_ATOM), lambda i: (i, 0)),
            pl.BlockSpec((EMB_EDGE, EMB_ATOM), lambda i: (0, 0)),
            pl.BlockSpec((EMB_ATOM, EMB_EDGE), lambda i: (0, 0)),
        ],
        out_specs=[
            pl.BlockSpec((TN, EMB_ATOM), lambda i: (i, 0)),
            pl.BlockSpec((TN, EMB_EDGE), lambda i: (i, 0)),
        ],
        out_shape=[
            jax.ShapeDtypeStruct((NP, EMB_ATOM), jnp.float32),
            jax.ShapeDtypeStruct((NP, EMB_EDGE), jnp.float32),
        ],
    )(s_part, s_part, h, w2b, w3b)


def _tc_force_edge(m3, g, u8, w_f):
    def body(m_ref, gs_ref, gd_ref, u_ref, wf_ref, f_ref):
        m = m_ref[...] + _silu(gs_ref[...] + gd_ref[...])
        f = _dot(m, wf_ref[...])
        f_ref[...] = f * u_ref[...]

    return pl.pallas_call(
        body,
        grid=(GE,),
        in_specs=[
            pl.BlockSpec((TE, EMB_EDGE), lambda i: (i, 0)),
            pl.BlockSpec((TE, EMB_EDGE), lambda i: (i, 0)),
            pl.BlockSpec((TE, EMB_EDGE), lambda i: (i + GE, 0)),
            pl.BlockSpec((TE, 8), lambda i: (i, 0)),
            pl.BlockSpec((EMB_EDGE, 1), lambda i: (0, 0)),
        ],
        out_specs=[pl.BlockSpec((TE, 8), lambda i: (i, 0))],
        out_shape=[jax.ShapeDtypeStruct((EP, 8), jnp.float32)],
    )(m3, g, g, u8, w_f)[0]


def _tc_out(h, sf, w_e):
    def body(h_ref, f0_ref, f1_ref, we_ref, o_ref):
        e = _dot(h_ref[...], we_ref[...])
        f = f0_ref[0] + f1_ref[0]
        o_ref[:, 0:1] = e
        o_ref[:, 1:4] = f[:, 0:3]

    return pl.pallas_call(
        body,
        grid=(GN,),
        in_specs=[
            pl.BlockSpec((TN, EMB_ATOM), lambda i: (i, 0)),
            pl.BlockSpec((1, TN, 8), lambda i: (0, i, 0)),
            pl.BlockSpec((1, TN, 8), lambda i: (1, i, 0)),
            pl.BlockSpec((EMB_ATOM, 1), lambda i: (0, 0)),
        ],
        out_specs=[pl.BlockSpec((TN, 4), lambda i: (i, 0))],
        out_shape=[jax.ShapeDtypeStruct((NP, 4), jnp.float32)],
    )(h, sf, sf, w_e)[0]


def kernel(atomic_numbers, pos, edge_index, atom_emb, W_rbf, W_edge_in, W1, Ws, W2, W3, W_E, W_F):
    src = edge_index[0].astype(jnp.int32)
    dst = edge_index[1].astype(jnp.int32)
    pad_e = jnp.full((EP - E,), N, jnp.int32)
    srcp = jnp.concatenate([src, pad_e])
    dstp = jnp.concatenate([dst, pad_e])
    idx2 = jnp.concatenate([srcp, dstp])
    anp = jnp.concatenate(
        [atomic_numbers.astype(jnp.int32), jnp.zeros((65536 - N,), jnp.int32)])
    pos16 = jnp.zeros((NP, 16), jnp.float32).at[:N, :3].set(pos)
    zeros32 = jnp.zeros((NP, EMB_EDGE), jnp.float32)
    zeros8 = jnp.zeros((NP, 8), jnp.float32)
    wa = W_edge_in[:EMB_ATOM]
    wb = W_edge_in[EMB_ATOM:2 * EMB_ATOM]
    wc = W_edge_in[2 * EMB_ATOM:]

    h = _sc_gather(atom_emb, anp, EMB_ATOM)
    a0, b0 = _tc_atom_init(h, wa, wb)
    asrc = _sc_gather(a0, srcp, EMB_EDGE)
    bdst = _sc_gather(b0, dstp, EMB_EDGE)
    gpos = _sc_gather(pos16, idx2, 16)
    m, rbfh, u8 = _tc_edge_init(gpos, asrc, bdst, W_rbf, wc)

    g = None
    for blk in range(NUM_BLOCKS):
        m, m2 = _tc_edge_block(m, rbfh, W1[blk], Ws[blk], g)
        s_part = _sc_scatter_add(m2, dstp, zeros32, EMB_EDGE)
        h, p = _tc_atom_update(s_part, h, W2[blk], W3[blk])
        g = _sc_gather(p, idx2, EMB_EDGE)

    fe = _tc_force_edge(m, g, u8, W_F)
    sf = _sc_scatter_add(fe, dstp, zeros8, 8)
    out = _tc_out(h, sf, W_E)
    return out[:N]
